# baseline (speedup 1.0000x reference)
import functools

import jax
import jax.numpy as jnp
import numpy as np
from jax import lax
from jax.experimental import pallas as pl
from jax.experimental.pallas import tpu as pltpu
from jax.experimental.pallas import tpu_sc as plsc

RESOLUTION = 1000
D = 64
B = 16384
S = 26
N = B * S

NC = 2
NS = 16
NW = NC * NS
BPW = B // NW
PER_W = N // NW
LANES = 16

DT = D // 8
BT = B // 128
BT_W = BT // NW
NG = 128 // LANES

PE_STRIDE = RESOLUTION + 1
BUF_MINOR = 129


def _body(idx_hbm, peT_hbm, out_hbm, idx_v, pe_v, buf0, buf1,
          wsem0, wsem1):
    wid = lax.axis_index("s") * NC + lax.axis_index("c")
    base = wid * PER_W

    pltpu.sync_copy(peT_hbm, pe_v)
    pltpu.sync_copy(idx_hbm.at[pl.ds(base, PER_W)], idx_v.at[pl.ds(0, PER_W)])

    lanes = lax.iota(jnp.int32, LANES)
    lane_pe = lanes * PE_STRIDE
    lane_s = lanes * S
    idx_dl = lanes & 7
    idx_dts = [(lanes >> 3) + tp * 2 for tp in range(D // LANES)]

    def wait_write(buf, wsem):
        pltpu.make_async_copy(
            buf.at[:, :, pl.ds(0, 128)], out_hbm.at[0, :, 0], wsem
        ).wait()

    def s_tile(t, buf, wsem, unroll):
        del unroll
        bt_l = t // S
        s = t % S
        bt = wid * BT_W + bt_l
        colbase = (bt_l * 128) * S + s

        @plsc.parallel_loop(0, NG)
        def grp_fill(g):
            iv = plsc.load_gather(idx_v, [lane_s + (colbase + g * LANES * S)])
            for j in range(LANES):
                ib_vec = jnp.broadcast_to(iv[j], (LANES,))
                bl_vec = jnp.full((LANES,), 0, jnp.int32) + (g * LANES + j)
                for tp in range(D // LANES):
                    v = plsc.load_gather(
                        pe_v, [lane_pe + ib_vec + tp * LANES * PE_STRIDE]
                    )
                    plsc.store_scatter(buf, [idx_dts[tp], idx_dl, bl_vec], v)

        pltpu.async_copy(
            buf.at[:, :, pl.ds(0, 128)], out_hbm.at[s, :, bt], wsem
        )

    NT = BT_W * S
    bufs = (buf0, buf1)
    wsems = (wsem0, wsem1)
    NBUF = len(bufs)
    for j in range(NBUF):
        s_tile(j, bufs[j], wsems[j], unroll=False)

    def loop_body(g, carry):
        for j in range(NBUF):
            wait_write(bufs[j], wsems[j])
            s_tile(NBUF * g + j, bufs[j], wsems[j], unroll=False)
        return carry

    lax.fori_loop(1, NT // NBUF, loop_body, 0)
    for j in range(NBUF):
        wait_write(bufs[j], wsems[j])


@jax.jit
def _encode(idx_flat, peT_flat):
    mesh = plsc.VectorSubcoreMesh(
        core_axis_name="c", subcore_axis_name="s", num_cores=NC, num_subcores=NS
    )
    return pl.kernel(
        _body,
        out_type=jax.ShapeDtypeStruct((S, DT, BT, 8, 128), jnp.float32),
        mesh=mesh,
        scratch_types=[
            pltpu.VMEM((PER_W + LANES,), jnp.int32),
            pltpu.VMEM((PE_STRIDE * D,), jnp.float32),
            pltpu.VMEM((DT, 8, BUF_MINOR), jnp.float32),
            pltpu.VMEM((DT, 8, BUF_MINOR), jnp.float32),
            pltpu.SemaphoreType.DMA,
            pltpu.SemaphoreType.DMA,
        ],
        compiler_params=pltpu.CompilerParams(
            use_tc_tiling_on_sc=False, needs_layout_passes=False
        ),
    )(idx_flat, peT_flat)


def kernel(x, pe):
    xc = jnp.clip(x, 1.0 / RESOLUTION, None)
    idx = jnp.round(xc * RESOLUTION).astype(jnp.int32) - 1
    peT_flat = jnp.pad(pe.T, ((0, 0), (0, 1))).reshape(PE_STRIDE * D)
    out5d = _encode(idx.reshape(N), peT_flat)
    return out5d.transpose(2, 4, 0, 1, 3).reshape(B, S, D)


# speedup vs baseline: 1.8375x; 1.8375x over previous
import functools

import jax
import jax.numpy as jnp
import numpy as np
from jax import lax
from jax.experimental import pallas as pl
from jax.experimental.pallas import tpu as pltpu
from jax.experimental.pallas import tpu_sc as plsc

RESOLUTION = 1000
D = 64
B = 16384
S = 26
N = B * S

NC = 2
NS = 16
NW = NC * NS
BPW = B // NW
PER_W = N // NW
LANES = 16

DT = D // 8
BT = B // 128
BT_W = BT // NW
NG = 128 // LANES

PE_STRIDE = RESOLUTION + 1
BUF_MINOR = 129


def _body(idx_hbm, peT_hbm, out_hbm, idx_v, pe_v, buf0, buf1,
          wsem0, wsem1):
    wid = lax.axis_index("s") * NC + lax.axis_index("c")
    base = wid * PER_W

    pltpu.sync_copy(peT_hbm, pe_v)
    pltpu.sync_copy(idx_hbm.at[pl.ds(base, PER_W)], idx_v.at[pl.ds(0, PER_W)])

    lanes = lax.iota(jnp.int32, LANES)
    lane_pe = lanes * PE_STRIDE
    idx_dl = lanes & 7
    idx_dts = [(lanes >> 3) + tp * 2 for tp in range(D // LANES)]

    def wait_write(buf, wsem):
        pltpu.make_async_copy(
            buf.at[:, :, pl.ds(0, 128)], out_hbm.at[0, :, 0], wsem
        ).wait()

    def s_tile(t, buf, wsem, unroll):
        del unroll
        bt_l = t // S
        s = t % S
        bt = wid * BT_W + bt_l
        colbase = (bt_l * 128) * S + s

        @plsc.parallel_loop(0, 128)
        def col_fill(bl):
            iv = idx_v[pl.ds(colbase + bl * S, LANES)]
            ib_vec = jnp.broadcast_to(iv[0], (LANES,))
            bl_vec = jnp.full((LANES,), 0, jnp.int32) + bl
            for tp in range(D // LANES):
                v = plsc.load_gather(
                    pe_v, [lane_pe + ib_vec + tp * LANES * PE_STRIDE]
                )
                plsc.store_scatter(buf, [idx_dts[tp], idx_dl, bl_vec], v)

        pltpu.async_copy(
            buf.at[:, :, pl.ds(0, 128)], out_hbm.at[s, :, bt], wsem
        )

    NT = BT_W * S
    bufs = (buf0, buf1)
    wsems = (wsem0, wsem1)
    NBUF = len(bufs)
    for j in range(NBUF):
        s_tile(j, bufs[j], wsems[j], unroll=False)

    def loop_body(g, carry):
        for j in range(NBUF):
            wait_write(bufs[j], wsems[j])
            s_tile(NBUF * g + j, bufs[j], wsems[j], unroll=False)
        return carry

    lax.fori_loop(1, NT // NBUF, loop_body, 0)
    for j in range(NBUF):
        wait_write(bufs[j], wsems[j])


@jax.jit
def _encode(idx_flat, peT_flat):
    mesh = plsc.VectorSubcoreMesh(
        core_axis_name="c", subcore_axis_name="s", num_cores=NC, num_subcores=NS
    )
    return pl.kernel(
        _body,
        out_type=jax.ShapeDtypeStruct((S, DT, BT, 8, 128), jnp.float32),
        mesh=mesh,
        scratch_types=[
            pltpu.VMEM((PER_W + LANES,), jnp.int32),
            pltpu.VMEM((PE_STRIDE * D,), jnp.float32),
            pltpu.VMEM((DT, 8, BUF_MINOR), jnp.float32),
            pltpu.VMEM((DT, 8, BUF_MINOR), jnp.float32),
            pltpu.SemaphoreType.DMA,
            pltpu.SemaphoreType.DMA,
        ],
        compiler_params=pltpu.CompilerParams(
            use_tc_tiling_on_sc=False, needs_layout_passes=False
        ),
    )(idx_flat, peT_flat)


def kernel(x, pe):
    xc = jnp.clip(x, 1.0 / RESOLUTION, None)
    idx = jnp.round(xc * RESOLUTION).astype(jnp.int32) - 1
    peT_flat = jnp.pad(pe.T, ((0, 0), (0, 1))).reshape(PE_STRIDE * D)
    out5d = _encode(idx.reshape(N), peT_flat)
    return out5d.transpose(2, 4, 0, 1, 3).reshape(B, S, D)
